# baseline (speedup 1.0000x reference)
import functools

import jax
import jax.numpy as jnp
from jax import lax
from jax.experimental import pallas as pl
from jax.experimental.pallas import tpu as pltpu
from jax.experimental.pallas import tpu_sc as plsc

VOCAB = 100000
B = 4096
S = 200
D = 64
NC, NS = 2, 16
NW = NC * NS
SEQ_PER_W = B // NW
HALF = 100
LANES = 16


def kernel(x, token_table, pos_table):
    x2 = x.astype(jnp.int32).reshape(B * S // HALF, HALF)
    mesh = plsc.VectorSubcoreMesh(core_axis_name="c", subcore_axis_name="s")

    @functools.partial(
        pl.kernel,
        out_type=jax.ShapeDtypeStruct((B * S * D,), jnp.float32),
        mesh=mesh,
        compiler_params=pltpu.CompilerParams(use_tc_tiling_on_sc=False),
        scratch_types=[
            pltpu.VMEM((2 * SEQ_PER_W, HALF), jnp.int32),
            pltpu.VMEM((S, D), jnp.float32),
            pltpu.VMEM((S, D), jnp.float32),
            pltpu.VMEM((S, D), jnp.float32),
            pltpu.VMEM((S * D,), jnp.float32),
            pltpu.VMEM((S * D,), jnp.float32),
            pltpu.SemaphoreType.DMA,
            pltpu.SemaphoreType.DMA,
            pltpu.SemaphoreType.DMA,
            pltpu.SemaphoreType.DMA,
        ],
    )
    def run(x_ref, tok_ref, pos_ref, out_ref,
            idx_v, pos_v, tok_v0, tok_v1, out_v0, out_v1,
            gsem0, gsem1, osem0, osem1):
        tok_v = (tok_v0, tok_v1)
        out_v = (out_v0, out_v1)
        gsem = (gsem0, gsem1)
        osem = (osem0, osem1)

        wid = lax.axis_index("s") * NC + lax.axis_index("c")
        base_seq = wid * SEQ_PER_W
        pltpu.sync_copy(pos_ref, pos_v)
        pltpu.sync_copy(x_ref.at[pl.ds(wid * 2 * SEQ_PER_W, 2 * SEQ_PER_W)], idx_v)

        def gather_starts(w, b):
            pltpu.async_copy(tok_ref.at[idx_v.at[2 * w]],
                             tok_v[b].at[pl.ds(0, HALF)], gsem[b])
            pltpu.async_copy(tok_ref.at[idx_v.at[2 * w + 1]],
                             tok_v[b].at[pl.ds(HALF, HALF)], gsem[b])

        def gather_waits(w, b):
            pltpu.make_async_copy(tok_ref.at[idx_v.at[2 * w]],
                                  tok_v[b].at[pl.ds(0, HALF)], gsem[b]).wait()
            pltpu.make_async_copy(tok_ref.at[idx_v.at[2 * w + 1]],
                                  tok_v[b].at[pl.ds(HALF, HALF)], gsem[b]).wait()

        gather_starts(0, 0)
        gather_starts(1, 1)

        @pl.loop(0, SEQ_PER_W, step=2)
        def _pair(g):
            for b in range(2):
                w = g + b
                gather_waits(w, b)

                @pl.when(w >= 2)
                def _():
                    pltpu.make_async_copy(
                        out_v[b],
                        out_ref.at[pl.ds((base_seq + w - 2) * S * D, S * D)],
                        osem[b]).wait()

                @pl.loop(0, S)
                def _row(r):
                    for j in range(D // LANES):
                        sl = pl.ds(j * LANES, LANES)
                        out_v[b][pl.ds(r * D + j * LANES, LANES)] = (
                            tok_v[b][r, sl] + pos_v[r, sl])

                pltpu.async_copy(out_v[b],
                                 out_ref.at[pl.ds((base_seq + w) * S * D, S * D)],
                                 osem[b])

                @pl.when(w + 2 < SEQ_PER_W)
                def _():
                    gather_starts(w + 2, b)

        for b in range(2):
            pltpu.make_async_copy(
                out_v[b],
                out_ref.at[pl.ds((base_seq + SEQ_PER_W - 2 + b) * S * D, S * D)],
                osem[b]).wait()

    return run(x2, token_table, pos_table).reshape(B, S, D)


# speedup vs baseline: 1.7304x; 1.7304x over previous
import functools

import jax
import jax.numpy as jnp
from jax import lax
from jax.experimental import pallas as pl
from jax.experimental.pallas import tpu as pltpu
from jax.experimental.pallas import tpu_sc as plsc

VOCAB = 100000
B = 4096
S = 200
D = 64
NC, NS = 2, 16
NW = NC * NS
SEQ_PER_W = B // NW
HALF = 100
LANES = 16


def kernel(x, token_table, pos_table):
    x2 = x.astype(jnp.int32).reshape(B * S // HALF, HALF)
    mesh = plsc.VectorSubcoreMesh(core_axis_name="c", subcore_axis_name="s")

    @functools.partial(
        pl.kernel,
        out_type=jax.ShapeDtypeStruct((B * S, 2 * D), jnp.float32),
        mesh=mesh,
        compiler_params=pltpu.CompilerParams(use_tc_tiling_on_sc=False),
        scratch_types=[
            pltpu.VMEM((2 * SEQ_PER_W, HALF), jnp.int32),
            pltpu.VMEM((S, D), jnp.float32),
            pltpu.VMEM((S, D), jnp.float32),
            pltpu.VMEM((S, D), jnp.float32),
            pltpu.VMEM((S, D), jnp.float32),
            pltpu.VMEM((S, D), jnp.float32),
            pltpu.SemaphoreType.DMA,
            pltpu.SemaphoreType.DMA,
            pltpu.SemaphoreType.DMA,
            pltpu.SemaphoreType.DMA,
        ],
    )
    def run(x_ref, tok_ref, pos_ref, out_ref,
            idx_v, pos_v, tok_v0, tok_v1, out_v0, out_v1,
            gsem0, gsem1, osem0, osem1):
        tok_v = (tok_v0, tok_v1)
        out_v = (out_v0, out_v1)
        gsem = (gsem0, gsem1)
        osem = (osem0, osem1)

        wid = lax.axis_index("s") * NC + lax.axis_index("c")
        base_seq = wid * SEQ_PER_W
        pltpu.sync_copy(pos_ref, pos_v)
        pltpu.sync_copy(x_ref.at[pl.ds(wid * 2 * SEQ_PER_W, 2 * SEQ_PER_W)], idx_v)

        def gather_starts(w, b):
            pltpu.async_copy(tok_ref.at[idx_v.at[2 * w]],
                             tok_v[b].at[pl.ds(0, HALF)], gsem[b])
            pltpu.async_copy(tok_ref.at[idx_v.at[2 * w + 1]],
                             tok_v[b].at[pl.ds(HALF, HALF)], gsem[b])

        def gather_waits(w, b):
            pltpu.make_async_copy(tok_ref.at[idx_v.at[2 * w]],
                                  tok_v[b].at[pl.ds(0, HALF)], gsem[b]).wait()
            pltpu.make_async_copy(tok_ref.at[idx_v.at[2 * w + 1]],
                                  tok_v[b].at[pl.ds(HALF, HALF)], gsem[b]).wait()

        gather_starts(0, 0)
        gather_starts(1, 1)

        @pl.loop(0, SEQ_PER_W, step=2)
        def _pair(g):
            for b in range(2):
                w = g + b
                gather_waits(w, b)

                @pl.when(w >= 2)
                def _():
                    pltpu.make_async_copy(
                        out_v[b],
                        out_ref.at[pl.ds((base_seq + w - 2) * S, S), pl.ds(0, D)],
                        osem[b]).wait()

                @pl.loop(0, S)
                def _row(r):
                    for j in range(D // LANES):
                        sl = pl.ds(j * LANES, LANES)
                        out_v[b][r, sl] = tok_v[b][r, sl] + pos_v[r, sl]

                pltpu.async_copy(
                    out_v[b],
                    out_ref.at[pl.ds((base_seq + w) * S, S), pl.ds(0, D)],
                    osem[b])

                @pl.when(w + 2 < SEQ_PER_W)
                def _():
                    gather_starts(w + 2, b)

        for b in range(2):
            pltpu.make_async_copy(
                out_v[b],
                out_ref.at[pl.ds((base_seq + SEQ_PER_W - 2 + b) * S, S),
                           pl.ds(0, D)],
                osem[b]).wait()

    out2d = run(x2, token_table, pos_table)
    return out2d[:, :D].reshape(B, S, D)
